# baseline (speedup 1.0000x reference)
import functools

import jax
import jax.numpy as jnp
from jax.experimental import pallas as pl
from jax.experimental.pallas import tpu as pltpu

REDUCTION_FACTOR = 1


def _scores_body(
    x0_ref, x1_ref, x2_ref, x3_ref, head_ref, focus_ref, acc_ref,
    *, num_heads, l_size
):
    g = pl.program_id(0)
    l = pl.program_id(1)
    nl = pl.num_programs(1)
    q = num_heads // 4
    for k, ref in enumerate((x0_ref, x1_ref, x2_ref, x3_ref)):
        x = ref[0]
        maxv = jnp.max(x, axis=1, keepdims=True)
        part = jnp.sum(maxv)
        idx = g + k * q

        @pl.when(l == 0)
        def _init(idx=idx, part=part):
            acc_ref[idx] = part

        @pl.when(l > 0)
        def _add(idx=idx, part=part):
            acc_ref[idx] += part

    @pl.when((g == q - 1) & (l == nl - 1))
    def _select():
        def body(i, carry):
            m, idx = carry
            v = acc_ref[i]
            better = v > m
            return jnp.where(better, v, m), jnp.where(better, i, idx)

        m, idx = jax.lax.fori_loop(
            0, num_heads, body, (jnp.float32(-jnp.inf), jnp.int32(0))
        )
        head_ref[0] = idx
        focus_ref[0] = m / l_size


def _durations_body(head_ref, x_ref, out_ref, *, t_size, num_chunks):
    del head_ref
    i = pl.program_id(0)
    x = x_ref[0]
    maxv = jnp.max(x, axis=1, keepdims=True)
    ti = jax.lax.broadcasted_iota(jnp.int32, x.shape, 1)
    am = jnp.min(jnp.where(x == maxv, ti, t_size), axis=1, keepdims=True)
    part = jnp.sum((am == ti).astype(jnp.int32), axis=0, keepdims=True)

    @pl.when(i == 0)
    def _init():
        out_ref[...] = part

    @pl.when(i > 0)
    def _acc():
        out_ref[...] += part


def kernel(att_ws):
    L = att_ws.shape[-2]
    T = att_ws.shape[-1]
    a = jnp.reshape(att_ws, (-1, L, T))
    H = a.shape[0]

    q = H // 4
    LB = L // 2
    head, focus = pl.pallas_call(
        functools.partial(_scores_body, num_heads=H, l_size=L),
        grid=(q, 2),
        in_specs=[
            pl.BlockSpec((1, LB, T), lambda h, l: (h, l, 0)),
            pl.BlockSpec((1, LB, T), lambda h, l: (h + q, l, 0)),
            pl.BlockSpec((1, LB, T), lambda h, l: (h + 2 * q, l, 0)),
            pl.BlockSpec((1, LB, T), lambda h, l: (h + 3 * q, l, 0)),
        ],
        out_specs=(
            pl.BlockSpec(memory_space=pltpu.SMEM),
            pl.BlockSpec(memory_space=pltpu.SMEM),
        ),
        out_shape=(
            jax.ShapeDtypeStruct((1,), jnp.int32),
            jax.ShapeDtypeStruct((1,), jnp.float32),
        ),
        scratch_shapes=[pltpu.SMEM((H,), jnp.float32)],
        compiler_params=pltpu.CompilerParams(
            dimension_semantics=("arbitrary", "arbitrary")
        ),
    )(a, a, a, a)

    CHUNK = 256
    NCH = L // CHUNK
    grid_spec = pltpu.PrefetchScalarGridSpec(
        num_scalar_prefetch=1,
        grid=(NCH,),
        in_specs=[pl.BlockSpec((1, CHUNK, T), lambda i, h: (h[0], i, 0))],
        out_specs=pl.BlockSpec((1, T), lambda i, h: (0, 0)),
    )
    durations2d = pl.pallas_call(
        functools.partial(_durations_body, t_size=T, num_chunks=NCH),
        grid_spec=grid_spec,
        out_shape=jax.ShapeDtypeStruct((1, T), jnp.int32),
    )(head, a)

    durations = durations2d[0] * REDUCTION_FACTOR
    return (durations, focus[0])
